# baseline (speedup 1.0000x reference)
import functools

import jax
import jax.numpy as jnp
from jax import lax
from jax.experimental import pallas as pl
from jax.experimental.pallas import tpu as pltpu
from jax.experimental.pallas import tpu_sc as plsc

_TOTAL = 32768
_MULS = (1, 3, 5, 7)
_PRIMES = (579983, 939997, 669989, 41491)
_BUCKETS = (20000, 20000, 10000, 500)

_NC, _NS, _L = 2, 16, 16
_NW = _NC * _NS
_CHUNK = _TOTAL // _NW
_VECS = _CHUNK // _L


def _mod_const(n, p):
    rp = jnp.float32(1.0) / jnp.float32(p)
    q = (n.astype(jnp.float32) * rp).astype(jnp.int32)
    r = n - q * jnp.int32(p)
    r = jnp.where(r < 0, r + jnp.int32(p), r)
    r = jnp.where(r >= jnp.int32(p), r - jnp.int32(p), r)
    return r


@functools.partial(
    pl.kernel,
    mesh=plsc.VectorSubcoreMesh(core_axis_name="c", subcore_axis_name="s"),
    out_type=[jax.ShapeDtypeStruct((_TOTAL,), jnp.uint32) for _ in range(4)],
    scratch_types=[pltpu.VMEM((_CHUNK,), jnp.int32)]
        + [pltpu.VMEM((_CHUNK,), jnp.uint32)] * 4,
)
def _multi_hash_sc(x_hbm, o0_hbm, o1_hbm, o2_hbm, o3_hbm,
                   xin, b0, b1, b2, b3):
    wid = lax.axis_index("s") * jnp.int32(_NC) + lax.axis_index("c")
    base = wid * jnp.int32(_CHUNK)
    pltpu.sync_copy(x_hbm.at[pl.ds(base, _CHUNK)], xin)
    bufs = (b0, b1, b2, b3)

    def body(i, _):
        sl = pl.ds(i * jnp.int32(_L), _L)
        v = xin[sl]
        for h in range(4):
            n = v * jnp.int32(_MULS[h]) if _MULS[h] != 1 else v
            r = _mod_const(n, _PRIMES[h])
            bufs[h][sl] = _mod_const(r, _BUCKETS[h]).astype(jnp.uint32)
        return 0

    lax.fori_loop(jnp.int32(0), jnp.int32(_VECS), body, 0)

    for h, o_hbm in enumerate((o0_hbm, o1_hbm, o2_hbm, o3_hbm)):
        pltpu.sync_copy(bufs[h], o_hbm.at[pl.ds(base, _CHUNK)])


def kernel(x):
    x32 = x.astype(jnp.int32)
    outs = _multi_hash_sc(x32)
    return tuple(o.astype(x.dtype) for o in outs)


# speedup vs baseline: 1.0807x; 1.0807x over previous
import functools

import jax
import jax.numpy as jnp
from jax import lax
from jax.experimental import pallas as pl
from jax.experimental.pallas import tpu as pltpu
from jax.experimental.pallas import tpu_sc as plsc

_TOTAL = 32768
_MULS = (1, 3, 5, 7)
_PRIMES = (579983, 939997, 669989, 41491)
_BUCKETS = (20000, 20000, 10000, 500)

_NC, _NS, _L = 2, 16, 16
_NW = _NC * _NS
_CHUNK = _TOTAL // _NW
_VECS = _CHUNK // _L


def _mod_const(n, p):
    rp = jnp.float32(1.0) / jnp.float32(p)
    q = (n.astype(jnp.float32) * rp).astype(jnp.int32)
    r = n - q * jnp.int32(p)
    r = jnp.where(r < 0, r + jnp.int32(p), r)
    r = jnp.where(r >= jnp.int32(p), r - jnp.int32(p), r)
    return r


@functools.partial(
    pl.kernel,
    mesh=plsc.VectorSubcoreMesh(core_axis_name="c", subcore_axis_name="s"),
    out_type=[jax.ShapeDtypeStruct((_TOTAL,), jnp.int32) for _ in range(4)],
    scratch_types=[pltpu.VMEM((_CHUNK,), jnp.int32)] * 5,
)
def _multi_hash_sc(x_hbm, o0_hbm, o1_hbm, o2_hbm, o3_hbm,
                   xin, b0, b1, b2, b3):
    wid = lax.axis_index("s") * jnp.int32(_NC) + lax.axis_index("c")
    base = wid * jnp.int32(_CHUNK)
    pltpu.sync_copy(x_hbm.at[pl.ds(base, _CHUNK)], xin)
    bufs = (b0, b1, b2, b3)

    def body(i, _):
        sl = pl.ds(i * jnp.int32(_L), _L)
        v = xin[sl]
        for h in range(4):
            n = v * jnp.int32(_MULS[h]) if _MULS[h] != 1 else v
            r = _mod_const(n, _PRIMES[h])
            bufs[h][sl] = _mod_const(r, _BUCKETS[h])
        return 0

    lax.fori_loop(jnp.int32(0), jnp.int32(_VECS), body, 0)

    for h, o_hbm in enumerate((o0_hbm, o1_hbm, o2_hbm, o3_hbm)):
        pltpu.sync_copy(bufs[h], o_hbm.at[pl.ds(base, _CHUNK)])


def kernel(x):
    x32 = x.astype(jnp.int32)
    outs = _multi_hash_sc(x32)
    return tuple(o.astype(x.dtype) for o in outs)


# speedup vs baseline: 1.0907x; 1.0093x over previous
import functools

import jax
import jax.numpy as jnp
from jax import lax
from jax.experimental import pallas as pl
from jax.experimental.pallas import tpu as pltpu
from jax.experimental.pallas import tpu_sc as plsc

_TOTAL = 32768
_MULS = (1, 3, 5, 7)
_PRIMES = (579983, 939997, 669989, 41491)
_BUCKETS = (20000, 20000, 10000, 500)

_NC, _NS, _L = 2, 16, 16
_NW = _NC * _NS
_CHUNK = _TOTAL // _NW
_VECS = _CHUNK // _L


import numpy as _np


def _recip_down(d):
    r = _np.float32(1.0) / _np.float32(d)
    for _ in range(3):
        r = _np.nextafter(r, _np.float32(0), dtype=_np.float32)
    return float(r)


def _mod_const(n, p):
    q = (n.astype(jnp.float32) * jnp.float32(_recip_down(p))).astype(jnp.int32)
    r = n - q * jnp.int32(p)
    r = jnp.where(r >= jnp.int32(p), r - jnp.int32(p), r)
    return r


@functools.partial(
    pl.kernel,
    mesh=plsc.VectorSubcoreMesh(core_axis_name="c", subcore_axis_name="s"),
    out_type=[jax.ShapeDtypeStruct((_TOTAL,), jnp.int32) for _ in range(4)],
    scratch_types=[pltpu.VMEM((_CHUNK,), jnp.int32)] * 5,
)
def _multi_hash_sc(x_hbm, o0_hbm, o1_hbm, o2_hbm, o3_hbm,
                   xin, b0, b1, b2, b3):
    wid = lax.axis_index("s") * jnp.int32(_NC) + lax.axis_index("c")
    base = wid * jnp.int32(_CHUNK)
    pltpu.sync_copy(x_hbm.at[pl.ds(base, _CHUNK)], xin)
    bufs = (b0, b1, b2, b3)

    def body(i, _):
        sl = pl.ds(i * jnp.int32(_L), _L)
        v = xin[sl]
        for h in range(4):
            n = v * jnp.int32(_MULS[h]) if _MULS[h] != 1 else v
            r = _mod_const(n, _PRIMES[h])
            bufs[h][sl] = _mod_const(r, _BUCKETS[h])
        return 0

    lax.fori_loop(jnp.int32(0), jnp.int32(_VECS), body, 0)

    for h, o_hbm in enumerate((o0_hbm, o1_hbm, o2_hbm, o3_hbm)):
        pltpu.sync_copy(bufs[h], o_hbm.at[pl.ds(base, _CHUNK)])


def kernel(x):
    x32 = x.astype(jnp.int32)
    outs = _multi_hash_sc(x32)
    return tuple(o.astype(x.dtype) for o in outs)
